# baseline (speedup 1.0000x reference)
import functools

import jax
import jax.numpy as jnp
from jax import lax
from jax.experimental import pallas as pl
from jax.experimental.pallas import tpu as pltpu
from jax.experimental.pallas import tpu_sc as plsc

N = 10000
E = 320000
D = 128
G = 16

NC = 2
NS = 16
NW = NC * NS
L = 16

NPT = 320
NPAD = NW * NPT

NCH_SCAN = 40
SCAN_CH = E // NCH_SCAN
DCH = 8192
CAP = 40 * DCH
GCH = 128

BR = 1024

_mesh = plsc.VectorSubcoreMesh(core_axis_name="c", subcore_axis_name="s")


def _worker_id():
    return lax.axis_index("s") * NC + lax.axis_index("c")


@functools.partial(
    pl.kernel,
    out_type=(
        jax.ShapeDtypeStruct((NW * CAP,), jnp.int32),
        jax.ShapeDtypeStruct((NW * CAP,), jnp.float32),
        jax.ShapeDtypeStruct((NW * 16,), jnp.int32),
        jax.ShapeDtypeStruct((NPAD,), jnp.float32),
    ),
    mesh=_mesh,
    scratch_types=[
        pltpu.VMEM((SCAN_CH,), jnp.int32),
        pltpu.VMEM((SCAN_CH,), jnp.int32),
        pltpu.VMEM((SCAN_CH,), jnp.int32),
        pltpu.VMEM((SCAN_CH,), jnp.int32),
        pltpu.VMEM((SCAN_CH,), jnp.float32),
        pltpu.VMEM((SCAN_CH,), jnp.float32),
        pltpu.VMEM((SCAN_CH + 16,), jnp.int32),
        pltpu.VMEM((SCAN_CH + 16,), jnp.int32),
        pltpu.VMEM((SCAN_CH + 16,), jnp.float32),
        pltpu.VMEM((SCAN_CH + 16,), jnp.float32),
        pltpu.VMEM((NPT,), jnp.float32),
        pltpu.VMEM((16,), jnp.int32),
        pltpu.SemaphoreType.DMA,
        pltpu.SemaphoreType.DMA,
        pltpu.SemaphoreType.DMA,
        pltpu.SemaphoreType.DMA,
    ],
    compiler_params=pltpu.CompilerParams(needs_layout_passes=False),
)
def _bucket(esrc, edst, ea, pk_out, w_out, cnt_out, deg_out,
            srcb0, srcb1, dstb0, dstb1, wb0, wb1,
            stg_pk0, stg_pk1, stg_w0, stg_w1, degacc, cnt_stg,
            se0, se1, sf0, sf1):
    srcb = (srcb0, srcb1)
    dstb = (dstb0, dstb1)
    wb = (wb0, wb1)
    stg_pk = (stg_pk0, stg_pk1)
    stg_w = (stg_w0, stg_w1)
    se = (se0, se1)
    sf = (sf0, sf1)

    wid = _worker_id()
    lo = wid * NPT
    iota = lax.iota(jnp.int32, 16)
    z16i = jnp.zeros((16,), jnp.int32)
    z16f = jnp.zeros((16,), jnp.float32)

    def zrod(k, _):
        degacc[pl.ds(k * 16, 16)] = z16f
        return 0

    lax.fori_loop(0, NPT // 16, zrod, 0)

    def in_slices(c):
        base = c * SCAN_CH
        return (esrc.at[pl.ds(base, SCAN_CH)],
                edst.at[pl.ds(base, SCAN_CH)],
                ea.at[pl.ds(base, SCAN_CH)])

    def start_load(c, b):
        s, d, w = in_slices(c)
        pltpu.async_copy(s, srcb[b], se[b])
        pltpu.async_copy(d, dstb[b], se[b])
        pltpu.async_copy(w, wb[b], se[b])

    def wait_load(c, b):
        s, d, w = in_slices(c)
        pltpu.make_async_copy(s, srcb[b], se[b]).wait()
        pltpu.make_async_copy(d, dstb[b], se[b]).wait()
        pltpu.make_async_copy(w, wb[b], se[b]).wait()

    def out_slices(gtot):
        off = pl.multiple_of(wid * CAP + gtot, 16)
        return (pk_out.at[pl.ds(off, SCAN_CH)],
                w_out.at[pl.ds(off, SCAN_CH)])

    def start_flush(gtot, b):
        po, wo = out_slices(gtot)
        pltpu.async_copy(stg_pk[b].at[pl.ds(0, SCAN_CH)], po, sf[b])
        pltpu.async_copy(stg_w[b].at[pl.ds(0, SCAN_CH)], wo, sf[b])

    def wait_flush(gtot, b):
        po, wo = out_slices(gtot)
        pltpu.make_async_copy(stg_pk[b].at[pl.ds(0, SCAN_CH)], po, sf[b]).wait()
        pltpu.make_async_copy(stg_w[b].at[pl.ds(0, SCAN_CH)], wo, sf[b]).wait()

    def scan_chunk(b):
        def grp(k, off):
            sl = pl.ds(k * 16, 16)
            dv = dstb[b][sl]
            m = (dv >= lo) & (dv < lo + NPT)
            wv = wb[b][sl]
            pk = ((dv - lo) << 16) | srcb[b][sl]
            csum = plsc.cumsum(m.astype(jnp.int32))
            idx = off + csum - 1
            plsc.store_scatter(stg_pk[b], [idx], pk, mask=m)
            plsc.store_scatter(stg_w[b], [idx], wv, mask=m)
            plsc.addupdate_scatter(degacc, [jnp.where(m, dv - lo, 0)],
                                   jnp.where(m, wv, 0.0))
            return off + csum[15]

        cnt_c = lax.fori_loop(0, SCAN_CH // 16, grp, 0)
        stg_pk[b][pl.ds(cnt_c, 16)] = z16i
        stg_w[b][pl.ds(cnt_c, 16)] = z16f
        return (cnt_c + 15) & ~15

    start_load(0, 0)
    start_load(1, 1)
    gtot = 0
    prev_flush = None
    for c in range(NCH_SCAN):
        b = c % 2
        wait_load(c, b)
        cnt_r = scan_chunk(b)
        if prev_flush is not None:
            wait_flush(prev_flush[0], prev_flush[1])
        start_flush(gtot, b)
        prev_flush = (gtot, b)
        gtot = gtot + cnt_r
        if c + 2 < NCH_SCAN:
            start_load(c + 2, b)
    wait_flush(prev_flush[0], prev_flush[1])

    total = gtot
    cnt_stg[...] = jnp.full((16,), total, jnp.int32)
    pltpu.sync_copy(cnt_stg, cnt_out.at[pl.ds(pl.multiple_of(wid * 16, 16), 16)])
    pltpu.sync_copy(degacc, deg_out.at[pl.ds(pl.multiple_of(wid * NPT, 16), NPT)])


@functools.partial(
    pl.kernel,
    out_type=jax.ShapeDtypeStruct((NPAD, D), jnp.float32),
    mesh=_mesh,
    scratch_types=[
        pltpu.VMEM((NPAD,), jnp.float32),
        pltpu.VMEM((16,), jnp.int32),
        pltpu.VMEM((GCH,), jnp.int32),
        pltpu.VMEM((GCH,), jnp.int32),
        pltpu.VMEM((GCH,), jnp.float32),
        pltpu.VMEM((GCH,), jnp.float32),
        pltpu.VMEM((GCH,), jnp.int32),
        pltpu.VMEM((GCH,), jnp.int32),
        pltpu.VMEM((GCH,), jnp.float32),
        pltpu.VMEM((GCH,), jnp.float32),
        pltpu.VMEM((GCH,), jnp.int32),
        pltpu.VMEM((GCH,), jnp.int32),
        pltpu.VMEM((GCH, D), jnp.float32),
        pltpu.VMEM((GCH, D), jnp.float32),
        pltpu.VMEM((NPT, D), jnp.float32),
        pltpu.SemaphoreType.DMA,
        pltpu.SemaphoreType.DMA,
        pltpu.SemaphoreType.DMA,
        pltpu.SemaphoreType.DMA,
    ],
    compiler_params=pltpu.CompilerParams(needs_layout_passes=False),
)
def _scatter(xw, dinv, pk_in, w_in, cnt_in, out,
             dinv_v, cnt_v, pkb0, pkb1, wb0, wb1, srcv0, srcv1,
             fv0, fv1, dlv0, dlv1, rows0, rows1, acc, sg0, sg1, sp0, sp1):
    pkb = (pkb0, pkb1)
    wb = (wb0, wb1)
    srcv = (srcv0, srcv1)
    fv = (fv0, fv1)
    dlv = (dlv0, dlv1)
    rows = (rows0, rows1)
    sg = (sg0, sg1)
    sp = (sp0, sp1)

    wid = _worker_id()
    lo = wid * NPT
    iota = lax.iota(jnp.int32, 16)
    z16f = jnp.zeros((16,), jnp.float32)

    pltpu.sync_copy(dinv, dinv_v)
    pltpu.sync_copy(cnt_in.at[pl.ds(pl.multiple_of(wid * 16, 16), 16)], cnt_v)
    total = cnt_v[...][0]

    def zro(r, _):
        for j in range(D // 16):
            acc[r, pl.ds(j * 16, 16)] = z16f
        return 0

    lax.fori_loop(0, NPT, zro, 0)

    nch = (total + GCH - 1) >> 7

    def pw_slices(g, b):
        off = pl.multiple_of(wid * CAP + g * GCH, 16)
        return (pk_in.at[pl.ds(off, GCH)], w_in.at[pl.ds(off, GCH)])

    def start_pw(g, b):
        pks, ws = pw_slices(g, b)
        pltpu.async_copy(pks, pkb[b], sp[b])
        pltpu.async_copy(ws, wb[b], sp[b])

    def wait_pw(g, b):
        pks, ws = pw_slices(g, b)
        pltpu.make_async_copy(pks, pkb[b], sp[b]).wait()
        pltpu.make_async_copy(ws, wb[b], sp[b]).wait()

    def pass1(g, b):
        base = g * GCH

        def p1(k, _):
            sl = pl.ds(k * 16, 16)
            valid = (base + k * 16 + iota) < total
            pk16 = jnp.where(valid, pkb[b][sl], 0)
            src16 = pk16 & 0xFFFF
            f16 = (jnp.where(valid, wb[b][sl], 0.0)
                   * plsc.load_gather(dinv_v, [src16]))
            dlv[b][sl] = lax.shift_right_logical(pk16, 16)
            srcv[b][sl] = src16
            fv[b][sl] = f16
            return 0

        lax.fori_loop(0, GCH // 16, p1, 0)

    def start_gather(b):
        pltpu.async_copy(xw.at[srcv[b]], rows[b], sg[b])

    def wait_gather(b):
        pltpu.make_async_copy(xw.at[srcv[b]], rows[b], sg[b]).wait()

    def pass2(b):
        def p2(k, _):
            sl = pl.ds(k * 16, 16)
            f16 = fv[b][sl]
            dl16 = dlv[b][sl]
            e16 = k * 16 + iota
            for j in range(D):
                jsplat = jnp.full((16,), j, jnp.int32)
                col = plsc.load_gather(rows[b], [e16, jsplat])
                plsc.addupdate_scatter(acc, [dl16, jsplat], f16 * col)
            return 0

        lax.fori_loop(0, GCH // 16, p2, 0)

    @pl.when(nch > 0)
    def _():
        start_pw(0, 0)
        wait_pw(0, 0)
        pass1(0, 0)
        start_gather(0)

        @pl.when(nch > 1)
        def _():
            start_pw(1, 1)

    def outer(g2, _):
        for b in (0, 1):
            g = g2 * 2 + b
            nb = 1 - b

            @pl.when(g < nch)
            def _(g=g, b=b, nb=nb):
                @pl.when(g + 1 < nch)
                def _():
                    wait_pw(g + 1, nb)

                    @pl.when(g + 2 < nch)
                    def _():
                        start_pw(g + 2, b)

                    pass1(g + 1, nb)
                    start_gather(nb)

                wait_gather(b)
                pass2(b)

        return 0

    lax.fori_loop(0, (nch + 1) >> 1, outer, 0)

    for off0, ln in ((0, GCH), (GCH, GCH), (2 * GCH, NPT - 2 * GCH)):
        pltpu.sync_copy(xw.at[pl.ds(lo + off0, ln)], rows0.at[pl.ds(0, ln)])

        def ep(k, _, off0=off0):
            d16 = dinv_v[pl.ds(lo + off0 + k * 16, 16)]
            for i in range(16):
                dsc = d16[i]
                r = off0 + k * 16 + i
                rr = k * 16 + i
                for j in range(D // 16):
                    cs = pl.ds(j * 16, 16)
                    acc[r, cs] = dsc * (acc[r, cs] + dsc * rows0[rr, cs])
            return 0

        lax.fori_loop(0, ln // 16, ep, 0)
    pltpu.sync_copy(acc, out.at[pl.ds(lo, NPT)])


def _dinv_body(deg_ref, o_ref):
    d = deg_ref[...] + 1.0
    o_ref[...] = jnp.where(d > 0.0, lax.rsqrt(jnp.maximum(d, 1e-12)), 0.0)


def _dinv(deg):
    deg2 = deg.reshape(NPAD // D, D)
    o = pl.pallas_call(
        _dinv_body,
        out_shape=jax.ShapeDtypeStruct((NPAD // D, D), jnp.float32),
    )(deg2)
    return o.reshape(NPAD)


def _mm_body(h_ref, w_ref, o_ref):
    o_ref[...] = jnp.dot(h_ref[...], w_ref[...],
                         preferred_element_type=jnp.float32)


def _mm0(h, w):
    return pl.pallas_call(
        _mm_body,
        grid=(NPAD // BR,),
        in_specs=[pl.BlockSpec((BR, D), lambda g: (g, 0)),
                  pl.BlockSpec((D, D), lambda g: (0, 0))],
        out_specs=pl.BlockSpec((BR, D), lambda g: (g, 0)),
        out_shape=jax.ShapeDtypeStruct((NPAD, D), jnp.float32),
    )(h, w)


def _step_body(s_ref, b_ref, w_ref, o_ref, *, act):
    h = s_ref[...] + b_ref[...]
    if act:
        h = jnp.tanh(h)
    o_ref[...] = jnp.dot(h, w_ref[...], preferred_element_type=jnp.float32)


def _step(scat, b2, w, act):
    return pl.pallas_call(
        functools.partial(_step_body, act=act),
        grid=(NPAD // BR,),
        in_specs=[pl.BlockSpec((BR, D), lambda g: (g, 0)),
                  pl.BlockSpec((1, D), lambda g: (0, 0)),
                  pl.BlockSpec((D, D), lambda g: (0, 0))],
        out_specs=pl.BlockSpec((BR, D), lambda g: (g, 0)),
        out_shape=jax.ShapeDtypeStruct((NPAD, D), jnp.float32),
    )(scat, b2, w)


_PBLK = 1000
_PNB = N // _PBLK


def _final_body(s_ref, b_ref, bt_ref, *fc_refs_and_out):
    fw = fc_refs_and_out[:6]
    fb = fc_refs_and_out[6:12]
    o_ref = fc_refs_and_out[12]
    psum, cnt = fc_refs_and_out[13], fc_refs_and_out[14]
    g = pl.program_id(0)

    @pl.when(g == 0)
    def _():
        psum[...] = jnp.zeros((G, D), jnp.float32)
        cnt[...] = jnp.zeros((G, D), jnp.float32)

    h6 = jnp.tanh(s_ref[...] + b_ref[...])
    bt = bt_ref[...].reshape(1, _PBLK)
    oh = (bt == lax.broadcasted_iota(jnp.int32, (G, 1), 0)).astype(jnp.float32)
    psum[...] += jnp.dot(oh, h6, preferred_element_type=jnp.float32)
    cnt[...] += jnp.broadcast_to(
        jnp.sum(oh, axis=1, keepdims=True), (G, D))

    @pl.when(g == _PNB - 1)
    def _():
        p = psum[...] / jnp.maximum(cnt[...], 1.0)
        for i in range(6):
            p = jnp.maximum(
                jnp.dot(p, fw[i][...], preferred_element_type=jnp.float32)
                + fb[i][...], 0.0)
        o_ref[...] = p


def _final(scat, b2, batch2, fws, fbs):
    return pl.pallas_call(
        _final_body,
        grid=(_PNB,),
        in_specs=[pl.BlockSpec((_PBLK, D), lambda g: (g, 0)),
                  pl.BlockSpec((1, D), lambda g: (0, 0)),
                  pl.BlockSpec((1, 1, _PBLK), lambda g: (g, 0, 0))]
                 + [pl.BlockSpec((D, D), lambda g: (0, 0))] * 6
                 + [pl.BlockSpec((1, D), lambda g: (0, 0))] * 6,
        out_specs=pl.BlockSpec((G, D), lambda g: (0, 0)),
        out_shape=jax.ShapeDtypeStruct((G, D), jnp.float32),
        scratch_shapes=[pltpu.VMEM((G, D), jnp.float32),
                        pltpu.VMEM((G, D), jnp.float32)],
    )(scat, b2, batch2, *fws, *fbs)


def kernel(x, edge_index, edge_attr, batch, params):
    xpad = jnp.concatenate(
        [x, jnp.zeros((NPAD - N, D), jnp.float32)], axis=0)

    pk, wgt, cnts, deg = _bucket(edge_index[0], edge_index[1], edge_attr)
    dinv = _dinv(deg)

    xw = _mm0(xpad, params["conv_W0"])
    scat = None
    for i in range(6):
        scat = _scatter(xw, dinv, pk, wgt, cnts)
        if i < 5:
            b2 = params[f"conv_b{i}"].reshape(1, D)
            xw = _step(scat, b2, params[f"conv_W{i + 1}"], act=(i != 2))

    b52 = params["conv_b5"].reshape(1, D)
    batch2 = batch.reshape(_PNB, 1, _PBLK)
    fws = [params[f"fc_W{i}"] for i in range(6)]
    fbs = [params[f"fc_b{i}"].reshape(1, D) for i in range(6)]
    return _final(scat, b52, batch2, fws, fbs)


# speedup vs baseline: 3.1560x; 3.1560x over previous
import functools

import jax
import jax.numpy as jnp
from jax import lax
from jax.experimental import pallas as pl
from jax.experimental.pallas import tpu as pltpu
from jax.experimental.pallas import tpu_sc as plsc

N = 10000
E = 320000
D = 128
G = 16

NC = 2
NS = 16
NW = NC * NS
L = 16

NPT = 320
NPAD = NW * NPT

NCH_SCAN = 40
SCAN_CH = E // NCH_SCAN
DCH = 8192
CAP = 40 * DCH
GCH = 128

BR = 1024

_mesh = plsc.VectorSubcoreMesh(core_axis_name="c", subcore_axis_name="s")


def _worker_id():
    return lax.axis_index("s") * NC + lax.axis_index("c")


@functools.partial(
    pl.kernel,
    out_type=(
        jax.ShapeDtypeStruct((NW * CAP,), jnp.int32),
        jax.ShapeDtypeStruct((NW * CAP,), jnp.float32),
        jax.ShapeDtypeStruct((NW * 16,), jnp.int32),
        jax.ShapeDtypeStruct((NPAD,), jnp.float32),
    ),
    mesh=_mesh,
    scratch_types=[
        pltpu.VMEM((SCAN_CH,), jnp.int32),
        pltpu.VMEM((SCAN_CH,), jnp.int32),
        pltpu.VMEM((SCAN_CH,), jnp.int32),
        pltpu.VMEM((SCAN_CH,), jnp.int32),
        pltpu.VMEM((SCAN_CH,), jnp.float32),
        pltpu.VMEM((SCAN_CH,), jnp.float32),
        pltpu.VMEM((SCAN_CH + 16,), jnp.int32),
        pltpu.VMEM((SCAN_CH + 16,), jnp.int32),
        pltpu.VMEM((SCAN_CH + 16,), jnp.float32),
        pltpu.VMEM((SCAN_CH + 16,), jnp.float32),
        pltpu.VMEM((NPT,), jnp.float32),
        pltpu.VMEM((16,), jnp.int32),
        pltpu.SemaphoreType.DMA,
        pltpu.SemaphoreType.DMA,
        pltpu.SemaphoreType.DMA,
        pltpu.SemaphoreType.DMA,
    ],
    compiler_params=pltpu.CompilerParams(needs_layout_passes=False),
)
def _bucket(esrc, edst, ea, pk_out, w_out, cnt_out, deg_out,
            srcb0, srcb1, dstb0, dstb1, wb0, wb1,
            stg_pk0, stg_pk1, stg_w0, stg_w1, degacc, cnt_stg,
            se0, se1, sf0, sf1):
    srcb = (srcb0, srcb1)
    dstb = (dstb0, dstb1)
    wb = (wb0, wb1)
    stg_pk = (stg_pk0, stg_pk1)
    stg_w = (stg_w0, stg_w1)
    se = (se0, se1)
    sf = (sf0, sf1)

    wid = _worker_id()
    lo = wid * NPT
    iota = lax.iota(jnp.int32, 16)
    z16i = jnp.zeros((16,), jnp.int32)
    z16f = jnp.zeros((16,), jnp.float32)

    def zrod(k, _):
        degacc[pl.ds(k * 16, 16)] = z16f
        return 0

    lax.fori_loop(0, NPT // 16, zrod, 0)

    def in_slices(c):
        base = c * SCAN_CH
        return (esrc.at[pl.ds(base, SCAN_CH)],
                edst.at[pl.ds(base, SCAN_CH)],
                ea.at[pl.ds(base, SCAN_CH)])

    def start_load(c, b):
        s, d, w = in_slices(c)
        pltpu.async_copy(s, srcb[b], se[b])
        pltpu.async_copy(d, dstb[b], se[b])
        pltpu.async_copy(w, wb[b], se[b])

    def wait_load(c, b):
        s, d, w = in_slices(c)
        pltpu.make_async_copy(s, srcb[b], se[b]).wait()
        pltpu.make_async_copy(d, dstb[b], se[b]).wait()
        pltpu.make_async_copy(w, wb[b], se[b]).wait()

    def out_slices(gtot):
        off = pl.multiple_of(wid * CAP + gtot, 16)
        return (pk_out.at[pl.ds(off, SCAN_CH)],
                w_out.at[pl.ds(off, SCAN_CH)])

    def start_flush(gtot, b):
        po, wo = out_slices(gtot)
        pltpu.async_copy(stg_pk[b].at[pl.ds(0, SCAN_CH)], po, sf[b])
        pltpu.async_copy(stg_w[b].at[pl.ds(0, SCAN_CH)], wo, sf[b])

    def wait_flush(gtot, b):
        po, wo = out_slices(gtot)
        pltpu.make_async_copy(stg_pk[b].at[pl.ds(0, SCAN_CH)], po, sf[b]).wait()
        pltpu.make_async_copy(stg_w[b].at[pl.ds(0, SCAN_CH)], wo, sf[b]).wait()

    def scan_chunk(b):
        def grp(k, off):
            sl = pl.ds(k * 16, 16)
            dv = dstb[b][sl]
            m = (dv >= lo) & (dv < lo + NPT)
            wv = wb[b][sl]
            pk = ((dv - lo) << 16) | srcb[b][sl]
            csum = plsc.cumsum(m.astype(jnp.int32))
            idx = off + csum - 1
            plsc.store_scatter(stg_pk[b], [idx], pk, mask=m)
            plsc.store_scatter(stg_w[b], [idx], wv, mask=m)
            plsc.addupdate_scatter(degacc, [jnp.where(m, dv - lo, 0)],
                                   jnp.where(m, wv, 0.0))
            return off + csum[15]

        cnt_c = lax.fori_loop(0, SCAN_CH // 16, grp, 0)
        stg_pk[b][pl.ds(cnt_c, 16)] = z16i
        stg_w[b][pl.ds(cnt_c, 16)] = z16f
        return (cnt_c + 15) & ~15

    start_load(0, 0)
    start_load(1, 1)
    gtot = 0
    prev_flush = None
    for c in range(NCH_SCAN):
        b = c % 2
        wait_load(c, b)
        cnt_r = scan_chunk(b)
        if prev_flush is not None:
            wait_flush(prev_flush[0], prev_flush[1])
        start_flush(gtot, b)
        prev_flush = (gtot, b)
        gtot = gtot + cnt_r
        if c + 2 < NCH_SCAN:
            start_load(c + 2, b)
    wait_flush(prev_flush[0], prev_flush[1])

    total = gtot
    cnt_stg[...] = jnp.full((16,), total, jnp.int32)
    pltpu.sync_copy(cnt_stg, cnt_out.at[pl.ds(pl.multiple_of(wid * 16, 16), 16)])
    pltpu.sync_copy(degacc, deg_out.at[pl.ds(pl.multiple_of(wid * NPT, 16), NPT)])


@functools.partial(
    pl.kernel,
    out_type=jax.ShapeDtypeStruct((NPAD, D), jnp.float32),
    mesh=_mesh,
    scratch_types=[
        pltpu.VMEM((NPAD,), jnp.float32),
        pltpu.VMEM((16,), jnp.int32),
        pltpu.VMEM((GCH,), jnp.int32),
        pltpu.VMEM((GCH,), jnp.int32),
        pltpu.VMEM((GCH,), jnp.float32),
        pltpu.VMEM((GCH,), jnp.float32),
        pltpu.VMEM((GCH,), jnp.int32),
        pltpu.VMEM((GCH,), jnp.int32),
        pltpu.VMEM((GCH,), jnp.float32),
        pltpu.VMEM((GCH,), jnp.float32),
        pltpu.VMEM((GCH,), jnp.int32),
        pltpu.VMEM((GCH,), jnp.int32),
        pltpu.VMEM((GCH, D), jnp.float32),
        pltpu.VMEM((GCH, D), jnp.float32),
        pltpu.VMEM((NPT, D), jnp.float32),
        pltpu.SemaphoreType.DMA,
        pltpu.SemaphoreType.DMA,
        pltpu.SemaphoreType.DMA,
        pltpu.SemaphoreType.DMA,
    ],
    compiler_params=pltpu.CompilerParams(needs_layout_passes=False),
)
def _scatter(xw, dinv, pk_in, w_in, cnt_in, out,
             dinv_v, cnt_v, pkb0, pkb1, wb0, wb1, srcv0, srcv1,
             fv0, fv1, dlv0, dlv1, rows0, rows1, acc, sg0, sg1, sp0, sp1):
    pkb = (pkb0, pkb1)
    wb = (wb0, wb1)
    srcv = (srcv0, srcv1)
    fv = (fv0, fv1)
    dlv = (dlv0, dlv1)
    rows = (rows0, rows1)
    sg = (sg0, sg1)
    sp = (sp0, sp1)

    wid = _worker_id()
    lo = wid * NPT
    iota = lax.iota(jnp.int32, 16)
    z16f = jnp.zeros((16,), jnp.float32)

    pltpu.sync_copy(dinv, dinv_v)
    pltpu.sync_copy(cnt_in.at[pl.ds(pl.multiple_of(wid * 16, 16), 16)], cnt_v)
    total = cnt_v[...][0]

    def zro(r, _):
        for j in range(D // 16):
            acc[r, pl.ds(j * 16, 16)] = z16f
        return 0

    lax.fori_loop(0, NPT, zro, 0)

    nch = (total + GCH - 1) >> 7

    def pw_slices(g, b):
        off = pl.multiple_of(wid * CAP + g * GCH, 16)
        return (pk_in.at[pl.ds(off, GCH)], w_in.at[pl.ds(off, GCH)])

    def start_pw(g, b):
        pks, ws = pw_slices(g, b)
        pltpu.async_copy(pks, pkb[b], sp[b])
        pltpu.async_copy(ws, wb[b], sp[b])

    def wait_pw(g, b):
        pks, ws = pw_slices(g, b)
        pltpu.make_async_copy(pks, pkb[b], sp[b]).wait()
        pltpu.make_async_copy(ws, wb[b], sp[b]).wait()

    def pass1(g, b):
        base = g * GCH

        def p1(k, _):
            sl = pl.ds(k * 16, 16)
            valid = (base + k * 16 + iota) < total
            pk16 = jnp.where(valid, pkb[b][sl], 0)
            src16 = pk16 & 0xFFFF
            f16 = (jnp.where(valid, wb[b][sl], 0.0)
                   * plsc.load_gather(dinv_v, [src16]))
            dlv[b][sl] = lax.shift_right_logical(pk16, 16)
            srcv[b][sl] = src16
            fv[b][sl] = f16
            return 0

        lax.fori_loop(0, GCH // 16, p1, 0)

    def start_gather(b):
        pltpu.async_copy(xw.at[srcv[b]], rows[b], sg[b])

    def wait_gather(b):
        pltpu.make_async_copy(xw.at[srcv[b]], rows[b], sg[b]).wait()

    def pass2(b):
        def p2(k, _):
            sl = pl.ds(k * 16, 16)
            f16 = fv[b][sl]
            dl16 = dlv[b][sl]
            for i in range(16):
                fs = f16[i]
                dl = dl16[i]
                e = k * 16 + i
                for j in range(D // 16):
                    cs = pl.ds(j * 16, 16)
                    plsc.addupdate(acc.at[dl, cs], fs * rows[b][e, cs])
            return 0

        lax.fori_loop(0, GCH // 16, p2, 0)

    @pl.when(nch > 0)
    def _():
        start_pw(0, 0)
        wait_pw(0, 0)
        pass1(0, 0)
        start_gather(0)

        @pl.when(nch > 1)
        def _():
            start_pw(1, 1)

    def outer(g2, _):
        for b in (0, 1):
            g = g2 * 2 + b
            nb = 1 - b

            @pl.when(g < nch)
            def _(g=g, b=b, nb=nb):
                @pl.when(g + 1 < nch)
                def _():
                    wait_pw(g + 1, nb)

                    @pl.when(g + 2 < nch)
                    def _():
                        start_pw(g + 2, b)

                    pass1(g + 1, nb)
                    start_gather(nb)

                wait_gather(b)
                pass2(b)

        return 0

    lax.fori_loop(0, (nch + 1) >> 1, outer, 0)

    for off0, ln in ((0, GCH), (GCH, GCH), (2 * GCH, NPT - 2 * GCH)):
        pltpu.sync_copy(xw.at[pl.ds(lo + off0, ln)], rows0.at[pl.ds(0, ln)])

        def ep(k, _, off0=off0):
            d16 = dinv_v[pl.ds(lo + off0 + k * 16, 16)]
            for i in range(16):
                dsc = d16[i]
                r = off0 + k * 16 + i
                rr = k * 16 + i
                for j in range(D // 16):
                    cs = pl.ds(j * 16, 16)
                    acc[r, cs] = dsc * (acc[r, cs] + dsc * rows0[rr, cs])
            return 0

        lax.fori_loop(0, ln // 16, ep, 0)
    pltpu.sync_copy(acc, out.at[pl.ds(lo, NPT)])


def _dinv_body(deg_ref, o_ref):
    d = deg_ref[...] + 1.0
    o_ref[...] = jnp.where(d > 0.0, lax.rsqrt(jnp.maximum(d, 1e-12)), 0.0)


def _dinv(deg):
    deg2 = deg.reshape(NPAD // D, D)
    o = pl.pallas_call(
        _dinv_body,
        out_shape=jax.ShapeDtypeStruct((NPAD // D, D), jnp.float32),
    )(deg2)
    return o.reshape(NPAD)


def _mm_body(h_ref, w_ref, o_ref):
    o_ref[...] = jnp.dot(h_ref[...], w_ref[...],
                         preferred_element_type=jnp.float32)


def _mm0(h, w):
    return pl.pallas_call(
        _mm_body,
        grid=(NPAD // BR,),
        in_specs=[pl.BlockSpec((BR, D), lambda g: (g, 0)),
                  pl.BlockSpec((D, D), lambda g: (0, 0))],
        out_specs=pl.BlockSpec((BR, D), lambda g: (g, 0)),
        out_shape=jax.ShapeDtypeStruct((NPAD, D), jnp.float32),
    )(h, w)


def _step_body(s_ref, b_ref, w_ref, o_ref, *, act):
    h = s_ref[...] + b_ref[...]
    if act:
        h = jnp.tanh(h)
    o_ref[...] = jnp.dot(h, w_ref[...], preferred_element_type=jnp.float32)


def _step(scat, b2, w, act):
    return pl.pallas_call(
        functools.partial(_step_body, act=act),
        grid=(NPAD // BR,),
        in_specs=[pl.BlockSpec((BR, D), lambda g: (g, 0)),
                  pl.BlockSpec((1, D), lambda g: (0, 0)),
                  pl.BlockSpec((D, D), lambda g: (0, 0))],
        out_specs=pl.BlockSpec((BR, D), lambda g: (g, 0)),
        out_shape=jax.ShapeDtypeStruct((NPAD, D), jnp.float32),
    )(scat, b2, w)


_PBLK = 1000
_PNB = N // _PBLK


def _final_body(s_ref, b_ref, bt_ref, *fc_refs_and_out):
    fw = fc_refs_and_out[:6]
    fb = fc_refs_and_out[6:12]
    o_ref = fc_refs_and_out[12]
    psum, cnt = fc_refs_and_out[13], fc_refs_and_out[14]
    g = pl.program_id(0)

    @pl.when(g == 0)
    def _():
        psum[...] = jnp.zeros((G, D), jnp.float32)
        cnt[...] = jnp.zeros((G, D), jnp.float32)

    h6 = jnp.tanh(s_ref[...] + b_ref[...])
    bt = bt_ref[...].reshape(1, _PBLK)
    oh = (bt == lax.broadcasted_iota(jnp.int32, (G, 1), 0)).astype(jnp.float32)
    psum[...] += jnp.dot(oh, h6, preferred_element_type=jnp.float32)
    cnt[...] += jnp.broadcast_to(
        jnp.sum(oh, axis=1, keepdims=True), (G, D))

    @pl.when(g == _PNB - 1)
    def _():
        p = psum[...] / jnp.maximum(cnt[...], 1.0)
        for i in range(6):
            p = jnp.maximum(
                jnp.dot(p, fw[i][...], preferred_element_type=jnp.float32)
                + fb[i][...], 0.0)
        o_ref[...] = p


def _final(scat, b2, batch2, fws, fbs):
    return pl.pallas_call(
        _final_body,
        grid=(_PNB,),
        in_specs=[pl.BlockSpec((_PBLK, D), lambda g: (g, 0)),
                  pl.BlockSpec((1, D), lambda g: (0, 0)),
                  pl.BlockSpec((1, 1, _PBLK), lambda g: (g, 0, 0))]
                 + [pl.BlockSpec((D, D), lambda g: (0, 0))] * 6
                 + [pl.BlockSpec((1, D), lambda g: (0, 0))] * 6,
        out_specs=pl.BlockSpec((G, D), lambda g: (0, 0)),
        out_shape=jax.ShapeDtypeStruct((G, D), jnp.float32),
        scratch_shapes=[pltpu.VMEM((G, D), jnp.float32),
                        pltpu.VMEM((G, D), jnp.float32)],
    )(scat, b2, batch2, *fws, *fbs)


def kernel(x, edge_index, edge_attr, batch, params):
    xpad = jnp.concatenate(
        [x, jnp.zeros((NPAD - N, D), jnp.float32)], axis=0)

    pk, wgt, cnts, deg = _bucket(edge_index[0], edge_index[1], edge_attr)
    dinv = _dinv(deg)

    xw = _mm0(xpad, params["conv_W0"])
    scat = None
    for i in range(6):
        scat = _scatter(xw, dinv, pk, wgt, cnts)
        if i < 5:
            b2 = params[f"conv_b{i}"].reshape(1, D)
            xw = _step(scat, b2, params[f"conv_W{i + 1}"], act=(i != 2))

    b52 = params["conv_b5"].reshape(1, D)
    batch2 = batch.reshape(_PNB, 1, _PBLK)
    fws = [params[f"fc_W{i}"] for i in range(6)]
    fbs = [params[f"fc_b{i}"].reshape(1, D) for i in range(6)]
    return _final(scat, b52, batch2, fws, fbs)


# speedup vs baseline: 3.2245x; 1.0217x over previous
import functools

import jax
import jax.numpy as jnp
from jax import lax
from jax.experimental import pallas as pl
from jax.experimental.pallas import tpu as pltpu
from jax.experimental.pallas import tpu_sc as plsc

N = 10000
E = 320000
D = 128
G = 16

NC = 2
NS = 16
NW = NC * NS
L = 16

NPT = 320
NPAD = NW * NPT

NCH_SCAN = 40
SCAN_CH = E // NCH_SCAN
DCH = 8192
CAP = 40 * DCH
GCH = 128

BR = 1024

_mesh = plsc.VectorSubcoreMesh(core_axis_name="c", subcore_axis_name="s")


def _worker_id():
    return lax.axis_index("s") * NC + lax.axis_index("c")


@functools.partial(
    pl.kernel,
    out_type=(
        jax.ShapeDtypeStruct((NW * CAP,), jnp.int32),
        jax.ShapeDtypeStruct((NW * CAP,), jnp.float32),
        jax.ShapeDtypeStruct((NW * 16,), jnp.int32),
        jax.ShapeDtypeStruct((NPAD,), jnp.float32),
    ),
    mesh=_mesh,
    scratch_types=[
        pltpu.VMEM((SCAN_CH,), jnp.int32),
        pltpu.VMEM((SCAN_CH,), jnp.int32),
        pltpu.VMEM((SCAN_CH,), jnp.int32),
        pltpu.VMEM((SCAN_CH,), jnp.int32),
        pltpu.VMEM((SCAN_CH,), jnp.float32),
        pltpu.VMEM((SCAN_CH,), jnp.float32),
        pltpu.VMEM((SCAN_CH + 16,), jnp.int32),
        pltpu.VMEM((SCAN_CH + 16,), jnp.int32),
        pltpu.VMEM((SCAN_CH + 16,), jnp.float32),
        pltpu.VMEM((SCAN_CH + 16,), jnp.float32),
        pltpu.VMEM((NPT,), jnp.float32),
        pltpu.VMEM((16,), jnp.int32),
        pltpu.SemaphoreType.DMA,
        pltpu.SemaphoreType.DMA,
        pltpu.SemaphoreType.DMA,
        pltpu.SemaphoreType.DMA,
    ],
    compiler_params=pltpu.CompilerParams(needs_layout_passes=False),
)
def _bucket(esrc, edst, ea, pk_out, w_out, cnt_out, deg_out,
            srcb0, srcb1, dstb0, dstb1, wb0, wb1,
            stg_pk0, stg_pk1, stg_w0, stg_w1, degacc, cnt_stg,
            se0, se1, sf0, sf1):
    srcb = (srcb0, srcb1)
    dstb = (dstb0, dstb1)
    wb = (wb0, wb1)
    stg_pk = (stg_pk0, stg_pk1)
    stg_w = (stg_w0, stg_w1)
    se = (se0, se1)
    sf = (sf0, sf1)

    wid = _worker_id()
    lo = wid * NPT
    iota = lax.iota(jnp.int32, 16)
    z16i = jnp.zeros((16,), jnp.int32)
    z16f = jnp.zeros((16,), jnp.float32)

    def zrod(k, _):
        degacc[pl.ds(k * 16, 16)] = z16f
        return 0

    lax.fori_loop(0, NPT // 16, zrod, 0)

    def in_slices(c):
        base = c * SCAN_CH
        return (esrc.at[pl.ds(base, SCAN_CH)],
                edst.at[pl.ds(base, SCAN_CH)],
                ea.at[pl.ds(base, SCAN_CH)])

    def start_load(c, b):
        s, d, w = in_slices(c)
        pltpu.async_copy(s, srcb[b], se[b])
        pltpu.async_copy(d, dstb[b], se[b])
        pltpu.async_copy(w, wb[b], se[b])

    def wait_load(c, b):
        s, d, w = in_slices(c)
        pltpu.make_async_copy(s, srcb[b], se[b]).wait()
        pltpu.make_async_copy(d, dstb[b], se[b]).wait()
        pltpu.make_async_copy(w, wb[b], se[b]).wait()

    def out_slices(gtot):
        off = pl.multiple_of(wid * CAP + gtot, 16)
        return (pk_out.at[pl.ds(off, SCAN_CH)],
                w_out.at[pl.ds(off, SCAN_CH)])

    def start_flush(gtot, b):
        po, wo = out_slices(gtot)
        pltpu.async_copy(stg_pk[b].at[pl.ds(0, SCAN_CH)], po, sf[b])
        pltpu.async_copy(stg_w[b].at[pl.ds(0, SCAN_CH)], wo, sf[b])

    def wait_flush(gtot, b):
        po, wo = out_slices(gtot)
        pltpu.make_async_copy(stg_pk[b].at[pl.ds(0, SCAN_CH)], po, sf[b]).wait()
        pltpu.make_async_copy(stg_w[b].at[pl.ds(0, SCAN_CH)], wo, sf[b]).wait()

    def scan_chunk(b):
        def grp(k, off):
            res = []
            for h in range(2):
                sl = pl.ds(k * 32 + h * 16, 16)
                dv = dstb[b][sl]
                m = (dv >= lo) & (dv < lo + NPT)
                wv = wb[b][sl]
                pk = ((dv - lo) << 16) | srcb[b][sl]
                csum = plsc.cumsum(m.astype(jnp.int32))
                res.append((m, wv, pk, csum, dv))
            for h in range(2):
                m, wv, pk, csum, dv = res[h]
                idx = off + csum - 1
                plsc.store_scatter(stg_pk[b], [idx], pk, mask=m)
                plsc.store_scatter(stg_w[b], [idx], wv, mask=m)
                plsc.addupdate_scatter(degacc, [jnp.where(m, dv - lo, 0)],
                                       jnp.where(m, wv, 0.0))
                off = off + csum[15]
            return off

        cnt_c = lax.fori_loop(0, SCAN_CH // 32, grp, 0)
        stg_pk[b][pl.ds(cnt_c, 16)] = z16i
        stg_w[b][pl.ds(cnt_c, 16)] = z16f
        return (cnt_c + 15) & ~15

    start_load(0, 0)
    start_load(1, 1)
    gtot = 0
    prev_flush = None
    for c in range(NCH_SCAN):
        b = c % 2
        wait_load(c, b)
        cnt_r = scan_chunk(b)
        if prev_flush is not None:
            wait_flush(prev_flush[0], prev_flush[1])
        start_flush(gtot, b)
        prev_flush = (gtot, b)
        gtot = gtot + cnt_r
        if c + 2 < NCH_SCAN:
            start_load(c + 2, b)
    wait_flush(prev_flush[0], prev_flush[1])

    total = gtot
    cnt_stg[...] = jnp.full((16,), total, jnp.int32)
    pltpu.sync_copy(cnt_stg, cnt_out.at[pl.ds(pl.multiple_of(wid * 16, 16), 16)])
    pltpu.sync_copy(degacc, deg_out.at[pl.ds(pl.multiple_of(wid * NPT, 16), NPT)])


@functools.partial(
    pl.kernel,
    out_type=jax.ShapeDtypeStruct((NPAD, D), jnp.float32),
    mesh=_mesh,
    scratch_types=[
        pltpu.VMEM((NPAD,), jnp.float32),
        pltpu.VMEM((16,), jnp.int32),
        pltpu.VMEM((GCH,), jnp.int32),
        pltpu.VMEM((GCH,), jnp.int32),
        pltpu.VMEM((GCH,), jnp.float32),
        pltpu.VMEM((GCH,), jnp.float32),
        pltpu.VMEM((GCH,), jnp.int32),
        pltpu.VMEM((GCH,), jnp.int32),
        pltpu.VMEM((GCH,), jnp.float32),
        pltpu.VMEM((GCH,), jnp.float32),
        pltpu.VMEM((GCH,), jnp.int32),
        pltpu.VMEM((GCH,), jnp.int32),
        pltpu.VMEM((GCH, D), jnp.float32),
        pltpu.VMEM((GCH, D), jnp.float32),
        pltpu.VMEM((NPT, D), jnp.float32),
        pltpu.SemaphoreType.DMA,
        pltpu.SemaphoreType.DMA,
        pltpu.SemaphoreType.DMA,
        pltpu.SemaphoreType.DMA,
    ],
    compiler_params=pltpu.CompilerParams(needs_layout_passes=False),
)
def _scatter(xw, dinv, pk_in, w_in, cnt_in, out,
             dinv_v, cnt_v, pkb0, pkb1, wb0, wb1, srcv0, srcv1,
             fv0, fv1, dlv0, dlv1, rows0, rows1, acc, sg0, sg1, sp0, sp1):
    pkb = (pkb0, pkb1)
    wb = (wb0, wb1)
    srcv = (srcv0, srcv1)
    fv = (fv0, fv1)
    dlv = (dlv0, dlv1)
    rows = (rows0, rows1)
    sg = (sg0, sg1)
    sp = (sp0, sp1)

    wid = _worker_id()
    lo = wid * NPT
    iota = lax.iota(jnp.int32, 16)
    z16f = jnp.zeros((16,), jnp.float32)

    pltpu.sync_copy(dinv, dinv_v)
    pltpu.sync_copy(cnt_in.at[pl.ds(pl.multiple_of(wid * 16, 16), 16)], cnt_v)
    total = cnt_v[...][0]

    def zro(r, _):
        for j in range(D // 16):
            acc[r, pl.ds(j * 16, 16)] = z16f
        return 0

    lax.fori_loop(0, NPT, zro, 0)

    nch = (total + GCH - 1) >> 7

    def pw_slices(g, b):
        off = pl.multiple_of(wid * CAP + g * GCH, 16)
        return (pk_in.at[pl.ds(off, GCH)], w_in.at[pl.ds(off, GCH)])

    def start_pw(g, b):
        pks, ws = pw_slices(g, b)
        pltpu.async_copy(pks, pkb[b], sp[b])
        pltpu.async_copy(ws, wb[b], sp[b])

    def wait_pw(g, b):
        pks, ws = pw_slices(g, b)
        pltpu.make_async_copy(pks, pkb[b], sp[b]).wait()
        pltpu.make_async_copy(ws, wb[b], sp[b]).wait()

    def pass1(g, b):
        base = g * GCH

        def p1(k, _):
            sl = pl.ds(k * 16, 16)
            valid = (base + k * 16 + iota) < total
            pk16 = jnp.where(valid, pkb[b][sl], 0)
            src16 = pk16 & 0xFFFF
            f16 = (jnp.where(valid, wb[b][sl], 0.0)
                   * plsc.load_gather(dinv_v, [src16]))
            dlv[b][sl] = lax.shift_right_logical(pk16, 16)
            srcv[b][sl] = src16
            fv[b][sl] = f16
            return 0

        lax.fori_loop(0, GCH // 16, p1, 0)

    def start_gather(b):
        pltpu.async_copy(xw.at[srcv[b]], rows[b], sg[b])

    def wait_gather(b):
        pltpu.make_async_copy(xw.at[srcv[b]], rows[b], sg[b]).wait()

    def pass2(b):
        def p2(k, _):
            sl = pl.ds(k * 16, 16)
            f16 = fv[b][sl]
            dl16 = dlv[b][sl]
            fs_l = [f16[i] for i in range(16)]
            dl_l = [dl16[i] for i in range(16)]
            for i in range(16):
                e = k * 16 + i
                for j in range(D // 16):
                    cs = pl.ds(j * 16, 16)
                    plsc.addupdate(acc.at[dl_l[i], cs],
                                   fs_l[i] * rows[b][e, cs])
            return 0

        lax.fori_loop(0, GCH // 16, p2, 0)

    @pl.when(nch > 0)
    def _():
        start_pw(0, 0)
        wait_pw(0, 0)
        pass1(0, 0)
        start_gather(0)

        @pl.when(nch > 1)
        def _():
            start_pw(1, 1)

    def outer(g2, _):
        for b in (0, 1):
            g = g2 * 2 + b
            nb = 1 - b

            @pl.when(g < nch)
            def _(g=g, b=b, nb=nb):
                @pl.when(g + 1 < nch)
                def _():
                    wait_pw(g + 1, nb)

                    @pl.when(g + 2 < nch)
                    def _():
                        start_pw(g + 2, b)

                    pass1(g + 1, nb)
                    start_gather(nb)

                wait_gather(b)
                pass2(b)

        return 0

    lax.fori_loop(0, (nch + 1) >> 1, outer, 0)

    for off0, ln in ((0, GCH), (GCH, GCH), (2 * GCH, NPT - 2 * GCH)):
        pltpu.sync_copy(xw.at[pl.ds(lo + off0, ln)], rows0.at[pl.ds(0, ln)])

        def ep(k, _, off0=off0):
            d16 = dinv_v[pl.ds(lo + off0 + k * 16, 16)]
            for i in range(16):
                dsc = d16[i]
                r = off0 + k * 16 + i
                rr = k * 16 + i
                for j in range(D // 16):
                    cs = pl.ds(j * 16, 16)
                    acc[r, cs] = dsc * (acc[r, cs] + dsc * rows0[rr, cs])
            return 0

        lax.fori_loop(0, ln // 16, ep, 0)
    pltpu.sync_copy(acc, out.at[pl.ds(lo, NPT)])


def _dinv_body(deg_ref, o_ref):
    d = deg_ref[...] + 1.0
    o_ref[...] = jnp.where(d > 0.0, lax.rsqrt(jnp.maximum(d, 1e-12)), 0.0)


def _dinv(deg):
    deg2 = deg.reshape(NPAD // D, D)
    o = pl.pallas_call(
        _dinv_body,
        out_shape=jax.ShapeDtypeStruct((NPAD // D, D), jnp.float32),
    )(deg2)
    return o.reshape(NPAD)


def _mm_body(h_ref, w_ref, o_ref):
    o_ref[...] = jnp.dot(h_ref[...], w_ref[...],
                         preferred_element_type=jnp.float32)


def _mm0(h, w):
    return pl.pallas_call(
        _mm_body,
        grid=(NPAD // BR,),
        in_specs=[pl.BlockSpec((BR, D), lambda g: (g, 0)),
                  pl.BlockSpec((D, D), lambda g: (0, 0))],
        out_specs=pl.BlockSpec((BR, D), lambda g: (g, 0)),
        out_shape=jax.ShapeDtypeStruct((NPAD, D), jnp.float32),
    )(h, w)


def _step_body(s_ref, b_ref, w_ref, o_ref, *, act):
    h = s_ref[...] + b_ref[...]
    if act:
        h = jnp.tanh(h)
    o_ref[...] = jnp.dot(h, w_ref[...], preferred_element_type=jnp.float32)


def _step(scat, b2, w, act):
    return pl.pallas_call(
        functools.partial(_step_body, act=act),
        grid=(NPAD // BR,),
        in_specs=[pl.BlockSpec((BR, D), lambda g: (g, 0)),
                  pl.BlockSpec((1, D), lambda g: (0, 0)),
                  pl.BlockSpec((D, D), lambda g: (0, 0))],
        out_specs=pl.BlockSpec((BR, D), lambda g: (g, 0)),
        out_shape=jax.ShapeDtypeStruct((NPAD, D), jnp.float32),
    )(scat, b2, w)


_PBLK = 1000
_PNB = N // _PBLK


def _final_body(s_ref, b_ref, bt_ref, *fc_refs_and_out):
    fw = fc_refs_and_out[:6]
    fb = fc_refs_and_out[6:12]
    o_ref = fc_refs_and_out[12]
    psum, cnt = fc_refs_and_out[13], fc_refs_and_out[14]
    g = pl.program_id(0)

    @pl.when(g == 0)
    def _():
        psum[...] = jnp.zeros((G, D), jnp.float32)
        cnt[...] = jnp.zeros((G, D), jnp.float32)

    h6 = jnp.tanh(s_ref[...] + b_ref[...])
    bt = bt_ref[...].reshape(1, _PBLK)
    oh = (bt == lax.broadcasted_iota(jnp.int32, (G, 1), 0)).astype(jnp.float32)
    psum[...] += jnp.dot(oh, h6, preferred_element_type=jnp.float32)
    cnt[...] += jnp.broadcast_to(
        jnp.sum(oh, axis=1, keepdims=True), (G, D))

    @pl.when(g == _PNB - 1)
    def _():
        p = psum[...] / jnp.maximum(cnt[...], 1.0)
        for i in range(6):
            p = jnp.maximum(
                jnp.dot(p, fw[i][...], preferred_element_type=jnp.float32)
                + fb[i][...], 0.0)
        o_ref[...] = p


def _final(scat, b2, batch2, fws, fbs):
    return pl.pallas_call(
        _final_body,
        grid=(_PNB,),
        in_specs=[pl.BlockSpec((_PBLK, D), lambda g: (g, 0)),
                  pl.BlockSpec((1, D), lambda g: (0, 0)),
                  pl.BlockSpec((1, 1, _PBLK), lambda g: (g, 0, 0))]
                 + [pl.BlockSpec((D, D), lambda g: (0, 0))] * 6
                 + [pl.BlockSpec((1, D), lambda g: (0, 0))] * 6,
        out_specs=pl.BlockSpec((G, D), lambda g: (0, 0)),
        out_shape=jax.ShapeDtypeStruct((G, D), jnp.float32),
        scratch_shapes=[pltpu.VMEM((G, D), jnp.float32),
                        pltpu.VMEM((G, D), jnp.float32)],
    )(scat, b2, batch2, *fws, *fbs)


def kernel(x, edge_index, edge_attr, batch, params):
    xpad = jnp.concatenate(
        [x, jnp.zeros((NPAD - N, D), jnp.float32)], axis=0)

    pk, wgt, cnts, deg = _bucket(edge_index[0], edge_index[1], edge_attr)
    dinv = _dinv(deg)

    xw = _mm0(xpad, params["conv_W0"])
    scat = None
    for i in range(6):
        scat = _scatter(xw, dinv, pk, wgt, cnts)
        if i < 5:
            b2 = params[f"conv_b{i}"].reshape(1, D)
            xw = _step(scat, b2, params[f"conv_W{i + 1}"], act=(i != 2))

    b52 = params["conv_b5"].reshape(1, D)
    batch2 = batch.reshape(_PNB, 1, _PBLK)
    fws = [params[f"fc_W{i}"] for i in range(6)]
    fbs = [params[f"fc_b{i}"].reshape(1, D) for i in range(6)]
    return _final(scat, b52, batch2, fws, fbs)
